# Initial kernel scaffold; baseline (speedup 1.0000x reference)
import jax
import jax.numpy as jnp
from jax.experimental import pallas as pl


def kernel(x, edge_index, edge_attr, W, b):
    raise NotImplementedError("write your pallas kernel here")


# speedup vs baseline: 21.5614x; 21.5614x over previous
import dataclasses
import functools

import jax
import jax.numpy as jnp
from jax import lax
from jax.experimental import pallas as pl
from jax.experimental.pallas import tpu as pltpu
from jax.experimental.pallas import tpu_sc as plsc

def _sc_compiler_params():
    cp = pltpu.CompilerParams()
    if "needs_layout_passes" in pltpu.CompilerParams.__dataclass_fields__:
        cp = dataclasses.replace(cp, needs_layout_passes=False)
    return cp


_NC = 2
_NS = 16
_NW = _NC * _NS
_LANES = 16
_BLK = 128


def _sc_degree(dst2d, w2d, n):
    rpt = dst2d.shape[0] // _NW
    zc = (n // _NS) // 8 * 8
    ztail = n - _NS * zc
    mesh = plsc.VectorSubcoreMesh(core_axis_name="c", subcore_axis_name="s")

    @functools.partial(
        pl.kernel,
        out_type=jax.ShapeDtypeStruct((_NC * n,), jnp.float32),
        mesh=mesh,
        scratch_types=[
            pltpu.VMEM((rpt, _BLK), jnp.int32),
            pltpu.VMEM((rpt, _BLK), jnp.float32),
            pltpu.VMEM((zc,), jnp.float32),
            pltpu.VMEM_SHARED((n,), jnp.float32),
        ],
    )
    def deg_kernel(dst_hbm, w_hbm, out_hbm, idx_v, w_v, buf_v, deg_sh):
        c = lax.axis_index("c")
        s = lax.axis_index("s")
        wid = c * _NS + s
        z0 = s * zc

        @pl.loop(0, zc // _LANES)
        def _(k):
            buf_v[pl.ds(k * _LANES, _LANES)] = jnp.zeros((_LANES,), jnp.float32)

        pltpu.sync_copy(buf_v, deg_sh.at[pl.ds(z0, zc)])

        @pl.when(s == _NS - 1)
        def _():
            if ztail:
                pltpu.sync_copy(buf_v.at[pl.ds(0, ztail)],
                                deg_sh.at[pl.ds(_NS * zc, ztail)])

        plsc.subcore_barrier()

        base = wid * rpt
        pltpu.sync_copy(dst_hbm.at[pl.ds(base, rpt)], idx_v)
        pltpu.sync_copy(w_hbm.at[pl.ds(base, rpt)], w_v)

        @pl.loop(0, rpt)
        def _(j):
            pltpu.sync_copy(w_v.at[j], deg_sh.at[idx_v.at[j]], add=True)

        plsc.subcore_barrier()
        pltpu.sync_copy(deg_sh.at[pl.ds(z0, zc)], buf_v)
        pltpu.sync_copy(buf_v, out_hbm.at[pl.ds(c * n + z0, zc)])

        @pl.when(s == _NS - 1)
        def _():
            if ztail:
                pltpu.sync_copy(deg_sh.at[pl.ds(_NS * zc, ztail)],
                                buf_v.at[pl.ds(0, ztail)])
                pltpu.sync_copy(buf_v.at[pl.ds(0, ztail)],
                                out_hbm.at[pl.ds(c * n + _NS * zc, ztail)])

    return deg_kernel(dst2d, w2d)


def _sc_messages(hd, src2d, dst2d, w2d):
    n, d = hd.shape
    rpt = src2d.shape[0] // _NW
    zc = (n // _NS) // 8 * 8
    ztail = n - _NS * zc
    chunks = [(o, min(_BLK, zc - o)) for o in range(0, zc, _BLK)]
    mesh = plsc.VectorSubcoreMesh(core_axis_name="c", subcore_axis_name="s")

    @functools.partial(
        pl.kernel,
        out_type=jax.ShapeDtypeStruct((_NC, n, d), jnp.float32),
        mesh=mesh,
        compiler_params=_sc_compiler_params(),
        scratch_types=[
            pltpu.VMEM((rpt, _BLK), jnp.int32),
            pltpu.VMEM((rpt, _BLK), jnp.int32),
            pltpu.VMEM((rpt, _BLK), jnp.float32),
            pltpu.VMEM((_BLK, 128), jnp.float32),
            pltpu.VMEM_SHARED((n, 128), jnp.float32),
        ],
    )
    def msg_kernel(hd_hbm, src_hbm, dst_hbm, w_hbm, out_hbm,
                   src_v, dst_v, w_v, rows_v, acc_sh):
        c = lax.axis_index("c")
        s = lax.axis_index("s")
        wid = c * _NS + s
        r0 = s * zc

        @pl.loop(0, _BLK)
        def _(i):
            for q in range(128 // _LANES):
                rows_v[i, pl.ds(q * _LANES, _LANES)] = (
                    jnp.zeros((_LANES,), jnp.float32))

        for off, nr in chunks:
            pltpu.sync_copy(rows_v.at[pl.ds(0, nr)],
                            acc_sh.at[pl.ds(r0 + off, nr)])

        @pl.when(s == _NS - 1)
        def _():
            if ztail:
                pltpu.sync_copy(rows_v.at[pl.ds(0, ztail)],
                                acc_sh.at[pl.ds(_NS * zc, ztail)])

        plsc.subcore_barrier()

        base = wid * rpt
        pltpu.sync_copy(src_hbm.at[pl.ds(base, rpt)], src_v)
        pltpu.sync_copy(dst_hbm.at[pl.ds(base, rpt)], dst_v)
        pltpu.sync_copy(w_hbm.at[pl.ds(base, rpt)], w_v)

        @pl.loop(0, rpt)
        def _(j):
            pltpu.sync_copy(hd_hbm.at[src_v.at[j]], rows_v)

            @pl.loop(0, _BLK)
            def _(i):
                wv = plsc.load_gather(
                    w_v, [jnp.full((_LANES,), j, jnp.int32),
                          jnp.full((_LANES,), i, jnp.int32)])
                for q in range(128 // _LANES):
                    sl = (i, pl.ds(q * _LANES, _LANES))
                    rows_v[sl] = rows_v[sl] * wv

            pltpu.sync_copy(rows_v, acc_sh.at[dst_v.at[j]], add=True)

        plsc.subcore_barrier()
        for off, nr in chunks:
            pltpu.sync_copy(acc_sh.at[pl.ds(r0 + off, nr)],
                            rows_v.at[pl.ds(0, nr)])
            pltpu.sync_copy(rows_v.at[pl.ds(0, nr)],
                            out_hbm.at[c, pl.ds(r0 + off, nr)])

        @pl.when(s == _NS - 1)
        def _():
            if ztail:
                pltpu.sync_copy(acc_sh.at[pl.ds(_NS * zc, ztail)],
                                rows_v.at[pl.ds(0, ztail)])
                pltpu.sync_copy(rows_v.at[pl.ds(0, ztail)],
                                out_hbm.at[c, pl.ds(_NS * zc, ztail)])

    return msg_kernel(hd, src2d, dst2d, w2d)


def _tc_matmul(x, W):
    n, d = x.shape
    blk = 1000

    def body(x_ref, w_ref, o_ref):
        o_ref[...] = jnp.dot(x_ref[...], w_ref[...],
                             preferred_element_type=jnp.float32)

    return pl.pallas_call(
        body,
        grid=(n // blk,),
        in_specs=[pl.BlockSpec((blk, d), lambda i: (i, 0)),
                  pl.BlockSpec((d, d), lambda i: (0, 0))],
        out_specs=pl.BlockSpec((blk, d), lambda i: (i, 0)),
        out_shape=jax.ShapeDtypeStruct((n, d), jnp.float32),
    )(x, W)


def _tc_scale(h, deg3):
    n, d = h.shape
    blk = 1000

    def body(h_ref, g_ref, o_ref):
        dg = g_ref[0] + g_ref[1] + 1.0
        dinv = jnp.where(dg > 0, lax.rsqrt(dg), 0.0)
        o_ref[...] = h_ref[...] * dinv

    return pl.pallas_call(
        body,
        grid=(n // blk,),
        in_specs=[pl.BlockSpec((blk, d), lambda i: (i, 0)),
                  pl.BlockSpec((2, blk, 1), lambda i: (0, i, 0))],
        out_specs=pl.BlockSpec((blk, d), lambda i: (i, 0)),
        out_shape=jax.ShapeDtypeStruct((n, d), jnp.float32),
    )(h, deg3)


def _tc_combine(A2, h, deg3, b):
    n, d = h.shape
    blk = 1000

    def body(a_ref, h_ref, g_ref, b_ref, o_ref):
        dg = g_ref[0] + g_ref[1] + 1.0
        dinv = jnp.where(dg > 0, lax.rsqrt(dg), 0.0)
        agg = (a_ref[0] + a_ref[1]) * dinv
        o_ref[...] = agg + h_ref[...] * (dinv * dinv) + b_ref[...]

    return pl.pallas_call(
        body,
        grid=(n // blk,),
        in_specs=[pl.BlockSpec((2, blk, d), lambda i: (0, i, 0)),
                  pl.BlockSpec((blk, d), lambda i: (i, 0)),
                  pl.BlockSpec((2, blk, 1), lambda i: (0, i, 0)),
                  pl.BlockSpec((1, d), lambda i: (0, 0))],
        out_specs=pl.BlockSpec((blk, d), lambda i: (i, 0)),
        out_shape=jax.ShapeDtypeStruct((n, d), jnp.float32),
    )(A2, h, deg3, b)


def kernel(x, edge_index, edge_attr, W, b):
    n, d = x.shape
    e = edge_attr.shape[0]
    src = edge_index[0]
    dst = edge_index[1]
    w = edge_attr

    rpt = -(-e // (_NW * _BLK))
    rpt = -(-rpt // 8) * 8
    e_pad = rpt * _NW * _BLK
    pad = e_pad - e
    if pad:
        fill = (jnp.arange(pad, dtype=jnp.int32) % n)
        src = jnp.concatenate([src, fill])
        dst = jnp.concatenate([dst, fill])
        w = jnp.concatenate([w, jnp.zeros((pad,), jnp.float32)])
    src2d = src.reshape(-1, _BLK)
    dst2d = dst.reshape(-1, _BLK)
    w2d = w.reshape(-1, _BLK)

    deg2 = _sc_degree(dst2d, w2d, n)
    deg3 = deg2.reshape(2, n, 1)
    h = _tc_matmul(x, W)
    hd = _tc_scale(h, deg3)
    A2 = _sc_messages(hd, src2d, dst2d, w2d)
    out2d = _tc_combine(A2, h, deg3, b.reshape(1, d))

    seq = 8
    return jnp.transpose(out2d.reshape(n, seq, d // seq), (1, 0, 2))[None]


# speedup vs baseline: 32.5354x; 1.5090x over previous
import dataclasses
import functools

import jax
import jax.numpy as jnp
from jax import lax
from jax.experimental import pallas as pl
from jax.experimental.pallas import tpu as pltpu
from jax.experimental.pallas import tpu_sc as plsc

def _sc_compiler_params():
    cp = pltpu.CompilerParams()
    if "needs_layout_passes" in pltpu.CompilerParams.__dataclass_fields__:
        cp = dataclasses.replace(cp, needs_layout_passes=False)
    return cp


_NC = 2
_NS = 16
_NW = _NC * _NS
_LANES = 16
_BLK = 128


def _sc_degree(dst2d, w2d, n):
    rpt = dst2d.shape[0] // _NW
    zc = (n // _NS) // 8 * 8
    ztail = n - _NS * zc
    mesh = plsc.VectorSubcoreMesh(core_axis_name="c", subcore_axis_name="s")

    @functools.partial(
        pl.kernel,
        out_type=jax.ShapeDtypeStruct((_NC * n,), jnp.float32),
        mesh=mesh,
        scratch_types=[
            pltpu.VMEM((rpt, _BLK), jnp.int32),
            pltpu.VMEM((rpt, _BLK), jnp.float32),
            pltpu.VMEM((zc,), jnp.float32),
            pltpu.VMEM_SHARED((n,), jnp.float32),
        ],
    )
    def deg_kernel(dst_hbm, w_hbm, out_hbm, idx_v, w_v, buf_v, deg_sh):
        c = lax.axis_index("c")
        s = lax.axis_index("s")
        wid = c * _NS + s
        z0 = s * zc

        @pl.loop(0, zc // _LANES)
        def _(k):
            buf_v[pl.ds(k * _LANES, _LANES)] = jnp.zeros((_LANES,), jnp.float32)

        pltpu.sync_copy(buf_v, deg_sh.at[pl.ds(z0, zc)])

        @pl.when(s == _NS - 1)
        def _():
            if ztail:
                pltpu.sync_copy(buf_v.at[pl.ds(0, ztail)],
                                deg_sh.at[pl.ds(_NS * zc, ztail)])

        plsc.subcore_barrier()

        base = wid * rpt
        pltpu.sync_copy(dst_hbm.at[pl.ds(base, rpt)], idx_v)
        pltpu.sync_copy(w_hbm.at[pl.ds(base, rpt)], w_v)

        @pl.loop(0, rpt)
        def _(j):
            pltpu.sync_copy(w_v.at[j], deg_sh.at[idx_v.at[j]], add=True)

        plsc.subcore_barrier()
        pltpu.sync_copy(deg_sh.at[pl.ds(z0, zc)], buf_v)
        pltpu.sync_copy(buf_v, out_hbm.at[pl.ds(c * n + z0, zc)])

        @pl.when(s == _NS - 1)
        def _():
            if ztail:
                pltpu.sync_copy(deg_sh.at[pl.ds(_NS * zc, ztail)],
                                buf_v.at[pl.ds(0, ztail)])
                pltpu.sync_copy(buf_v.at[pl.ds(0, ztail)],
                                out_hbm.at[pl.ds(c * n + _NS * zc, ztail)])

    return deg_kernel(dst2d, w2d)


def _sc_messages(hd, src1d, dst1d, w2d):
    n, d = hd.shape
    rpt = w2d.shape[0] // _NW
    zc = (n // _NS) // 8 * 8
    ztail = n - _NS * zc
    chunks = [(o, min(_BLK, zc - o)) for o in range(0, zc, _BLK)]
    mesh = plsc.VectorSubcoreMesh(core_axis_name="c", subcore_axis_name="s")

    nbuf = 2
    assert rpt % nbuf == 0

    scratch = [
        pltpu.VMEM((rpt, _BLK), jnp.float32),
        pltpu.VMEM((2, _BLK), jnp.int32),
        pltpu.VMEM((4, _BLK), jnp.int32),
    ]
    scratch += [pltpu.VMEM((_BLK, 128), jnp.float32) for _ in range(nbuf)]
    scratch += [pltpu.SemaphoreType.DMA for _ in range(2 * nbuf + 6)]
    scratch += [pltpu.VMEM_SHARED((n, 128), jnp.float32)]

    @functools.partial(
        pl.kernel,
        out_type=jax.ShapeDtypeStruct((_NC, n, d), jnp.float32),
        mesh=mesh,
        compiler_params=_sc_compiler_params(),
        scratch_types=scratch,
    )
    def msg_kernel(hd_hbm, src_hbm, dst_hbm, w_hbm, out_hbm,
                   w_v, srcw, dstw, *rest):
        bufs = rest[:nbuf]
        gsem = rest[nbuf:2 * nbuf]
        ssem = rest[2 * nbuf:3 * nbuf]
        sisem = rest[3 * nbuf:3 * nbuf + 2]
        disem = rest[3 * nbuf + 2:3 * nbuf + 6]
        acc_sh = rest[3 * nbuf + 6]
        c = lax.axis_index("c")
        s = lax.axis_index("s")
        wid = c * _NS + s
        r0 = s * zc
        base = wid * rpt

        def src_dma(j, p):
            return pltpu.make_async_copy(
                src_hbm.at[pl.ds((base + j) * _BLK, _BLK)],
                srcw.at[p], sisem[p])

        def dst_dma(j, p):
            return pltpu.make_async_copy(
                dst_hbm.at[pl.ds((base + j) * _BLK, _BLK)],
                dstw.at[p], disem[p])

        def gather_start(b, p):
            pltpu.async_copy(hd_hbm.at[srcw.at[p]], bufs[b], gsem[b])

        def gather_wait(b, p):
            pltpu.make_async_copy(hd_hbm.at[srcw.at[p]], bufs[b],
                                  gsem[b]).wait()

        def scatter_start(b, p):
            pltpu.async_copy(bufs[b], acc_sh.at[dstw.at[p]], ssem[b],
                             add=True)

        def scatter_wait(b, p):
            pltpu.make_async_copy(bufs[b], acc_sh.at[dstw.at[p]],
                                  ssem[b]).wait()

        @pl.loop(0, _BLK)
        def _(i):
            for q in range(128 // _LANES):
                bufs[0][i, pl.ds(q * _LANES, _LANES)] = (
                    jnp.zeros((_LANES,), jnp.float32))

        for off, nr in chunks:
            pltpu.sync_copy(bufs[0].at[pl.ds(0, nr)],
                            acc_sh.at[pl.ds(r0 + off, nr)])

        @pl.when(s == _NS - 1)
        def _():
            if ztail:
                pltpu.sync_copy(bufs[0].at[pl.ds(0, ztail)],
                                acc_sh.at[pl.ds(_NS * zc, ztail)])

        plsc.subcore_barrier()

        pltpu.sync_copy(w_hbm.at[pl.ds(base, rpt)], w_v)

        def scale_rows(m, b, lo, hi):
            @pl.loop(lo, hi, step=_LANES)
            def _(i0):
                wrow = w_v[m, pl.ds(i0, _LANES)]
                for u in range(_LANES):
                    wv = jnp.full((_LANES,), wrow[u])
                    for q in range(128 // _LANES):
                        sl = (i0 + u, pl.ds(q * _LANES, _LANES))
                        bufs[b][sl] = bufs[b][sl] * wv

        pltpu.sync_copy(src_hbm.at[pl.ds(base * _BLK, _BLK)], srcw.at[0])
        gather_start(0, 0)
        src_dma(1, 1).start()
        dst_dma(0, 0).start()

        @pl.loop(0, rpt, step=4)
        def _(j):
            for u in range(4):
                m = j + u
                b = u % nbuf
                ob = (b + 1) % nbuf
                gather_wait(b, u % 2)

                @pl.when(m + 1 < rpt)
                def _():
                    dst_dma(m + 1, (u + 1) % 4).start()

                @pl.when(m + 2 < rpt)
                def _():
                    src_dma(m + 2, u % 2).start()

                scale_rows(m, b, 0, _BLK // 2)

                @pl.when(m - 1 >= 0)
                def _():
                    scatter_wait(ob, (u + 3) % 4)

                @pl.when(m + 1 < rpt)
                def _():
                    src_dma(m + 1, (u + 1) % 2).wait()
                    gather_start(ob, (u + 1) % 2)

                scale_rows(m, b, _BLK // 2, _BLK)
                dst_dma(m, u).wait()
                scatter_start(b, u)

        scatter_wait((rpt - 1) % nbuf, (rpt - 1) % 4)

        plsc.subcore_barrier()
        for off, nr in chunks:
            pltpu.sync_copy(acc_sh.at[pl.ds(r0 + off, nr)],
                            bufs[0].at[pl.ds(0, nr)])
            pltpu.sync_copy(bufs[0].at[pl.ds(0, nr)],
                            out_hbm.at[c, pl.ds(r0 + off, nr)])

        @pl.when(s == _NS - 1)
        def _():
            if ztail:
                pltpu.sync_copy(acc_sh.at[pl.ds(_NS * zc, ztail)],
                                bufs[0].at[pl.ds(0, ztail)])
                pltpu.sync_copy(bufs[0].at[pl.ds(0, ztail)],
                                out_hbm.at[c, pl.ds(_NS * zc, ztail)])

    return msg_kernel(hd, src1d, dst1d, w2d)


def _tc_matmul(x, W):
    n, d = x.shape
    blk = 1000

    def body(x_ref, w_ref, o_ref):
        o_ref[...] = jnp.dot(x_ref[...], w_ref[...],
                             preferred_element_type=jnp.float32)

    return pl.pallas_call(
        body,
        grid=(n // blk,),
        in_specs=[pl.BlockSpec((blk, d), lambda i: (i, 0)),
                  pl.BlockSpec((d, d), lambda i: (0, 0))],
        out_specs=pl.BlockSpec((blk, d), lambda i: (i, 0)),
        out_shape=jax.ShapeDtypeStruct((n, d), jnp.float32),
    )(x, W)


def _tc_scale(h, deg3):
    n, d = h.shape
    blk = 1000

    def body(h_ref, g_ref, o_ref):
        dg = g_ref[0] + g_ref[1] + 1.0
        dinv = jnp.where(dg > 0, lax.rsqrt(dg), 0.0)
        o_ref[...] = h_ref[...] * dinv

    return pl.pallas_call(
        body,
        grid=(n // blk,),
        in_specs=[pl.BlockSpec((blk, d), lambda i: (i, 0)),
                  pl.BlockSpec((2, blk, 1), lambda i: (0, i, 0))],
        out_specs=pl.BlockSpec((blk, d), lambda i: (i, 0)),
        out_shape=jax.ShapeDtypeStruct((n, d), jnp.float32),
    )(h, deg3)


def _tc_combine(A2, h, deg3, b):
    n, d = h.shape
    blk = 1000

    def body(a_ref, h_ref, g_ref, b_ref, o_ref):
        dg = g_ref[0] + g_ref[1] + 1.0
        dinv = jnp.where(dg > 0, lax.rsqrt(dg), 0.0)
        agg = (a_ref[0] + a_ref[1]) * dinv
        o_ref[...] = agg + h_ref[...] * (dinv * dinv) + b_ref[...]

    return pl.pallas_call(
        body,
        grid=(n // blk,),
        in_specs=[pl.BlockSpec((2, blk, d), lambda i: (0, i, 0)),
                  pl.BlockSpec((blk, d), lambda i: (i, 0)),
                  pl.BlockSpec((2, blk, 1), lambda i: (0, i, 0)),
                  pl.BlockSpec((1, d), lambda i: (0, 0))],
        out_specs=pl.BlockSpec((blk, d), lambda i: (i, 0)),
        out_shape=jax.ShapeDtypeStruct((n, d), jnp.float32),
    )(A2, h, deg3, b)


def kernel(x, edge_index, edge_attr, W, b):
    n, d = x.shape
    e = edge_attr.shape[0]
    src = edge_index[0]
    dst = edge_index[1]
    w = edge_attr

    rpt = -(-e // (_NW * _BLK))
    rpt = -(-rpt // 8) * 8
    e_pad = rpt * _NW * _BLK
    pad = e_pad - e
    if pad:
        fill = (jnp.arange(pad, dtype=jnp.int32) % n)
        src = jnp.concatenate([src, fill])
        dst = jnp.concatenate([dst, fill])
        w = jnp.concatenate([w, jnp.zeros((pad,), jnp.float32)])
    src2d = src.reshape(-1, _BLK)
    dst2d = dst.reshape(-1, _BLK)
    w2d = w.reshape(-1, _BLK)

    deg2 = _sc_degree(dst2d, w2d, n)
    deg3 = deg2.reshape(2, n, 1)
    h = _tc_matmul(x, W)
    hd = _tc_scale(h, deg3)
    A2 = _sc_messages(hd, src, dst, w2d)
    out2d = _tc_combine(A2, h, deg3, b.reshape(1, d))

    seq = 8
    return jnp.transpose(out2d.reshape(n, seq, d // seq), (1, 0, 2))[None]


# speedup vs baseline: 34.5000x; 1.0604x over previous
import dataclasses
import functools

import jax
import jax.numpy as jnp
from jax import lax
from jax.experimental import pallas as pl
from jax.experimental.pallas import tpu as pltpu
from jax.experimental.pallas import tpu_sc as plsc

def _sc_compiler_params():
    cp = pltpu.CompilerParams()
    if "needs_layout_passes" in pltpu.CompilerParams.__dataclass_fields__:
        cp = dataclasses.replace(cp, needs_layout_passes=False)
    return cp


_NC = 2
_NS = 16
_NW = _NC * _NS
_LANES = 16
_BLK = 128


def _sc_degree(dst2d, w2d, n):
    rpt = dst2d.shape[0] // _NW
    zc = (n // _NS) // 8 * 8
    ztail = n - _NS * zc
    mesh = plsc.VectorSubcoreMesh(core_axis_name="c", subcore_axis_name="s")

    @functools.partial(
        pl.kernel,
        out_type=jax.ShapeDtypeStruct((_NC * n,), jnp.float32),
        mesh=mesh,
        scratch_types=[
            pltpu.VMEM((rpt, _BLK), jnp.int32),
            pltpu.VMEM((rpt, _BLK), jnp.float32),
            pltpu.VMEM((zc,), jnp.float32),
            pltpu.VMEM_SHARED((n,), jnp.float32),
        ],
    )
    def deg_kernel(dst_hbm, w_hbm, out_hbm, idx_v, w_v, buf_v, deg_sh):
        c = lax.axis_index("c")
        s = lax.axis_index("s")
        wid = c * _NS + s
        z0 = s * zc

        @pl.loop(0, zc // _LANES)
        def _(k):
            buf_v[pl.ds(k * _LANES, _LANES)] = jnp.zeros((_LANES,), jnp.float32)

        pltpu.sync_copy(buf_v, deg_sh.at[pl.ds(z0, zc)])

        @pl.when(s == _NS - 1)
        def _():
            if ztail:
                pltpu.sync_copy(buf_v.at[pl.ds(0, ztail)],
                                deg_sh.at[pl.ds(_NS * zc, ztail)])

        plsc.subcore_barrier()

        base = wid * rpt
        pltpu.sync_copy(dst_hbm.at[pl.ds(base, rpt)], idx_v)
        pltpu.sync_copy(w_hbm.at[pl.ds(base, rpt)], w_v)

        @pl.loop(0, rpt)
        def _(j):
            pltpu.sync_copy(w_v.at[j], deg_sh.at[idx_v.at[j]], add=True)

        plsc.subcore_barrier()
        pltpu.sync_copy(deg_sh.at[pl.ds(z0, zc)], buf_v)
        pltpu.sync_copy(buf_v, out_hbm.at[pl.ds(c * n + z0, zc)])

        @pl.when(s == _NS - 1)
        def _():
            if ztail:
                pltpu.sync_copy(deg_sh.at[pl.ds(_NS * zc, ztail)],
                                buf_v.at[pl.ds(0, ztail)])
                pltpu.sync_copy(buf_v.at[pl.ds(0, ztail)],
                                out_hbm.at[pl.ds(c * n + _NS * zc, ztail)])

    return deg_kernel(dst2d, w2d)


def _sc_messages(hd, src1d, dst1d, w1d):
    n, d = hd.shape
    rpt = src1d.shape[0] // (_NW * _BLK)
    zc = (n // _NS) // 8 * 8
    ztail = n - _NS * zc
    chunks = [(o, min(_BLK, zc - o)) for o in range(0, zc, _BLK)]
    mesh = plsc.VectorSubcoreMesh(core_axis_name="c", subcore_axis_name="s")

    nbuf = 3
    step = 12
    assert rpt % step == 0

    scratch = [
        pltpu.VMEM((3, _BLK), jnp.int32),
        pltpu.VMEM((4, _BLK), jnp.int32),
        pltpu.VMEM((4, _BLK), jnp.float32),
    ]
    scratch += [pltpu.VMEM((_BLK, 128), jnp.float32) for _ in range(nbuf)]
    scratch += [pltpu.SemaphoreType.DMA for _ in range(nbuf + nbuf + 3 + 4 + 4)]
    scratch += [pltpu.VMEM_SHARED((n, 128), jnp.float32)]

    @functools.partial(
        pl.kernel,
        out_type=jax.ShapeDtypeStruct((_NC, n, d), jnp.float32),
        mesh=mesh,
        compiler_params=_sc_compiler_params(),
        scratch_types=scratch,
    )
    def msg_kernel(hd_hbm, src_hbm, dst_hbm, w_hbm, out_hbm,
                   srcw, dstw, ww, *rest):
        bufs = rest[:nbuf]
        gsem = rest[nbuf:2 * nbuf]
        ssem = rest[2 * nbuf:3 * nbuf]
        sisem = rest[3 * nbuf:3 * nbuf + 3]
        disem = rest[3 * nbuf + 3:3 * nbuf + 7]
        wsem = rest[3 * nbuf + 7:3 * nbuf + 11]
        acc_sh = rest[3 * nbuf + 11]
        c = lax.axis_index("c")
        s = lax.axis_index("s")
        wid = c * _NS + s
        r0 = s * zc
        base = wid * rpt

        def src_dma(j, p):
            return pltpu.make_async_copy(
                src_hbm.at[pl.ds((base + j) * _BLK, _BLK)],
                srcw.at[p], sisem[p])

        def dst_dma(j, p):
            return pltpu.make_async_copy(
                dst_hbm.at[pl.ds((base + j) * _BLK, _BLK)],
                dstw.at[p], disem[p])

        def w_dma(j, p):
            return pltpu.make_async_copy(
                w_hbm.at[pl.ds((base + j) * _BLK, _BLK)],
                ww.at[p], wsem[p])

        def gather_start(b, p):
            pltpu.async_copy(hd_hbm.at[srcw.at[p]], bufs[b], gsem[b])

        def gather_wait(b, p):
            pltpu.make_async_copy(hd_hbm.at[srcw.at[p]], bufs[b],
                                  gsem[b]).wait()

        def scatter_start(b, p):
            pltpu.async_copy(bufs[b], acc_sh.at[dstw.at[p]], ssem[b],
                             add=True)

        def scatter_wait(b, p):
            pltpu.make_async_copy(bufs[b], acc_sh.at[dstw.at[p]],
                                  ssem[b]).wait()

        @pl.loop(0, _BLK)
        def _(i):
            for q in range(128 // _LANES):
                bufs[0][i, pl.ds(q * _LANES, _LANES)] = (
                    jnp.zeros((_LANES,), jnp.float32))

        for off, nr in chunks:
            pltpu.sync_copy(bufs[0].at[pl.ds(0, nr)],
                            acc_sh.at[pl.ds(r0 + off, nr)])

        @pl.when(s == _NS - 1)
        def _():
            if ztail:
                pltpu.sync_copy(bufs[0].at[pl.ds(0, ztail)],
                                acc_sh.at[pl.ds(_NS * zc, ztail)])

        plsc.subcore_barrier()

        def scale_rows(b, wp):
            @pl.loop(0, _BLK, step=_LANES)
            def _(i0):
                wrow = ww[wp, pl.ds(i0, _LANES)]
                for u in range(_LANES):
                    wv = jnp.full((_LANES,), wrow[u])
                    for q in range(128 // _LANES):
                        sl = (i0 + u, pl.ds(q * _LANES, _LANES))
                        bufs[b][sl] = bufs[b][sl] * wv

        pltpu.sync_copy(src_hbm.at[pl.ds(base * _BLK, _BLK)], srcw.at[0])
        gather_start(0, 0)
        src_dma(1, 1).start()
        dst_dma(0, 0).start()
        w_dma(0, 0).start()

        @pl.loop(0, rpt, step=step)
        def _(j):
            for u in range(step):
                m = j + u
                b = u % nbuf
                gather_wait(b, u % 3)

                @pl.when(m + 2 < rpt)
                def _():
                    src_dma(m + 2, (u + 2) % 3).start()

                @pl.when(m + 1 < rpt)
                def _():
                    dst_dma(m + 1, (u + 1) % 4).start()
                    w_dma(m + 1, (u + 1) % 4).start()

                @pl.when(m - 2 >= 0)
                def _():
                    scatter_wait((u + 1) % 3, (u + 2) % 4)

                @pl.when(m + 1 < rpt)
                def _():
                    src_dma(m + 1, (u + 1) % 3).wait()
                    gather_start((u + 1) % 3, (u + 1) % 3)

                w_dma(m, u % 4).wait()
                scale_rows(b, u % 4)

                dst_dma(m, u % 4).wait()
                scatter_start(b, u % 4)

        scatter_wait((rpt - 2) % nbuf, (rpt - 2) % 4)
        scatter_wait((rpt - 1) % nbuf, (rpt - 1) % 4)

        plsc.subcore_barrier()
        for off, nr in chunks:
            pltpu.sync_copy(acc_sh.at[pl.ds(r0 + off, nr)],
                            bufs[0].at[pl.ds(0, nr)])
            pltpu.sync_copy(bufs[0].at[pl.ds(0, nr)],
                            out_hbm.at[c, pl.ds(r0 + off, nr)])

        @pl.when(s == _NS - 1)
        def _():
            if ztail:
                pltpu.sync_copy(acc_sh.at[pl.ds(_NS * zc, ztail)],
                                bufs[0].at[pl.ds(0, ztail)])
                pltpu.sync_copy(bufs[0].at[pl.ds(0, ztail)],
                                out_hbm.at[c, pl.ds(_NS * zc, ztail)])

    return msg_kernel(hd, src1d, dst1d, w1d)


def _tc_matmul(x, W):
    n, d = x.shape
    blk = 1000

    def body(x_ref, w_ref, o_ref):
        o_ref[...] = jnp.dot(x_ref[...], w_ref[...],
                             preferred_element_type=jnp.float32)

    return pl.pallas_call(
        body,
        grid=(n // blk,),
        in_specs=[pl.BlockSpec((blk, d), lambda i: (i, 0)),
                  pl.BlockSpec((d, d), lambda i: (0, 0))],
        out_specs=pl.BlockSpec((blk, d), lambda i: (i, 0)),
        out_shape=jax.ShapeDtypeStruct((n, d), jnp.float32),
    )(x, W)


def _tc_scale(h, deg3):
    n, d = h.shape
    blk = 1000

    def body(h_ref, g_ref, o_ref):
        dg = g_ref[0] + g_ref[1] + 1.0
        dinv = jnp.where(dg > 0, lax.rsqrt(dg), 0.0)
        o_ref[...] = h_ref[...] * dinv

    return pl.pallas_call(
        body,
        grid=(n // blk,),
        in_specs=[pl.BlockSpec((blk, d), lambda i: (i, 0)),
                  pl.BlockSpec((2, blk, 1), lambda i: (0, i, 0))],
        out_specs=pl.BlockSpec((blk, d), lambda i: (i, 0)),
        out_shape=jax.ShapeDtypeStruct((n, d), jnp.float32),
    )(h, deg3)


def _tc_combine(A2, h, deg3, b):
    n, d = h.shape
    blk = 1000

    def body(a_ref, h_ref, g_ref, b_ref, o_ref):
        dg = g_ref[0] + g_ref[1] + 1.0
        dinv = jnp.where(dg > 0, lax.rsqrt(dg), 0.0)
        agg = (a_ref[0] + a_ref[1]) * dinv
        o_ref[...] = agg + h_ref[...] * (dinv * dinv) + b_ref[...]

    return pl.pallas_call(
        body,
        grid=(n // blk,),
        in_specs=[pl.BlockSpec((2, blk, d), lambda i: (0, i, 0)),
                  pl.BlockSpec((blk, d), lambda i: (i, 0)),
                  pl.BlockSpec((2, blk, 1), lambda i: (0, i, 0)),
                  pl.BlockSpec((1, d), lambda i: (0, 0))],
        out_specs=pl.BlockSpec((blk, d), lambda i: (i, 0)),
        out_shape=jax.ShapeDtypeStruct((n, d), jnp.float32),
    )(A2, h, deg3, b)


def kernel(x, edge_index, edge_attr, W, b):
    n, d = x.shape
    e = edge_attr.shape[0]
    src = edge_index[0]
    dst = edge_index[1]
    w = edge_attr

    def pad_edges(rows):
        tgt = rows * _NW * _BLK
        padn = tgt - e
        if not padn:
            return src, dst, w
        fill = jnp.arange(padn, dtype=jnp.int32) % n
        return (jnp.concatenate([src, fill]),
                jnp.concatenate([dst, fill]),
                jnp.concatenate([w, jnp.zeros((padn,), jnp.float32)]))

    rpt0 = -(-e // (_NW * _BLK))
    _, dstd, wd = pad_edges(-(-rpt0 // 8) * 8)
    srcm, dstm, wm = pad_edges(-(-rpt0 // 12) * 12)

    deg2 = _sc_degree(dstd.reshape(-1, _BLK), wd.reshape(-1, _BLK), n)
    deg3 = deg2.reshape(2, n, 1)
    h = _tc_matmul(x, W)
    hd = _tc_scale(h, deg3)
    A2 = _sc_messages(hd, srcm, dstm, wm)
    out2d = _tc_combine(A2, h, deg3, b.reshape(1, d))

    seq = 8
    return jnp.transpose(out2d.reshape(n, seq, d // seq), (1, 0, 2))[None]


# speedup vs baseline: 35.3908x; 1.0258x over previous
import dataclasses
import functools

import jax
import jax.numpy as jnp
from jax import lax
from jax.experimental import pallas as pl
from jax.experimental.pallas import tpu as pltpu
from jax.experimental.pallas import tpu_sc as plsc

def _sc_compiler_params():
    cp = pltpu.CompilerParams()
    if "needs_layout_passes" in pltpu.CompilerParams.__dataclass_fields__:
        cp = dataclasses.replace(cp, needs_layout_passes=False)
    return cp


_NC = 2
_NS = 16
_NW = _NC * _NS
_LANES = 16
_BLK = 128


def _sc_degree(dst2d, w2d, n):
    rpt = dst2d.shape[0] // _NW
    zc = (n // _NS) // 8 * 8
    ztail = n - _NS * zc
    mesh = plsc.VectorSubcoreMesh(core_axis_name="c", subcore_axis_name="s")

    @functools.partial(
        pl.kernel,
        out_type=jax.ShapeDtypeStruct((_NC * n,), jnp.float32),
        mesh=mesh,
        scratch_types=[
            pltpu.VMEM((rpt, _BLK), jnp.int32),
            pltpu.VMEM((rpt, _BLK), jnp.float32),
            pltpu.VMEM((zc,), jnp.float32),
            pltpu.VMEM_SHARED((n,), jnp.float32),
        ],
    )
    def deg_kernel(dst_hbm, w_hbm, out_hbm, idx_v, w_v, buf_v, deg_sh):
        c = lax.axis_index("c")
        s = lax.axis_index("s")
        wid = c * _NS + s
        z0 = s * zc

        @pl.loop(0, zc // _LANES)
        def _(k):
            buf_v[pl.ds(k * _LANES, _LANES)] = jnp.zeros((_LANES,), jnp.float32)

        pltpu.sync_copy(buf_v, deg_sh.at[pl.ds(z0, zc)])

        @pl.when(s == _NS - 1)
        def _():
            if ztail:
                pltpu.sync_copy(buf_v.at[pl.ds(0, ztail)],
                                deg_sh.at[pl.ds(_NS * zc, ztail)])

        plsc.subcore_barrier()

        base = wid * rpt
        pltpu.sync_copy(dst_hbm.at[pl.ds(base, rpt)], idx_v)
        pltpu.sync_copy(w_hbm.at[pl.ds(base, rpt)], w_v)

        @pl.loop(0, rpt)
        def _(j):
            pltpu.sync_copy(w_v.at[j], deg_sh.at[idx_v.at[j]], add=True)

        plsc.subcore_barrier()
        pltpu.sync_copy(deg_sh.at[pl.ds(z0, zc)], buf_v)
        pltpu.sync_copy(buf_v, out_hbm.at[pl.ds(c * n + z0, zc)])

        @pl.when(s == _NS - 1)
        def _():
            if ztail:
                pltpu.sync_copy(deg_sh.at[pl.ds(_NS * zc, ztail)],
                                buf_v.at[pl.ds(0, ztail)])
                pltpu.sync_copy(buf_v.at[pl.ds(0, ztail)],
                                out_hbm.at[pl.ds(c * n + _NS * zc, ztail)])

    return deg_kernel(dst2d, w2d)


def _sc_messages(hd, src1d, dst1d, w1d):
    n, d = hd.shape
    rpt = src1d.shape[0] // (_NW * _BLK)
    zc = (n // _NS) // 8 * 8
    ztail = n - _NS * zc
    chunks = [(o, min(_BLK, zc - o)) for o in range(0, zc, _BLK)]
    mesh = plsc.VectorSubcoreMesh(core_axis_name="c", subcore_axis_name="s")

    nbuf = 3
    step = 12
    assert rpt % step == 0

    scratch = [
        pltpu.VMEM((3, _BLK), jnp.int32),
        pltpu.VMEM((4, _BLK), jnp.int32),
        pltpu.VMEM((4, _BLK), jnp.float32),
    ]
    scratch += [pltpu.VMEM((_BLK, 128), jnp.float32) for _ in range(nbuf)]
    scratch += [pltpu.SemaphoreType.DMA for _ in range(nbuf + nbuf + 3 + 4 + 4)]
    scratch += [pltpu.VMEM_SHARED((n, 128), jnp.float32)]

    @functools.partial(
        pl.kernel,
        out_type=jax.ShapeDtypeStruct((_NC, n, d), jnp.float32),
        mesh=mesh,
        compiler_params=_sc_compiler_params(),
        scratch_types=scratch,
    )
    def msg_kernel(hd_hbm, src_hbm, dst_hbm, w_hbm, out_hbm,
                   srcw, dstw, ww, *rest):
        bufs = rest[:nbuf]
        gsem = rest[nbuf:2 * nbuf]
        ssem = rest[2 * nbuf:3 * nbuf]
        sisem = rest[3 * nbuf:3 * nbuf + 3]
        disem = rest[3 * nbuf + 3:3 * nbuf + 7]
        wsem = rest[3 * nbuf + 7:3 * nbuf + 11]
        acc_sh = rest[3 * nbuf + 11]
        c = lax.axis_index("c")
        s = lax.axis_index("s")
        wid = c * _NS + s
        r0 = s * zc
        base = wid * rpt

        def src_dma(j, p):
            return pltpu.make_async_copy(
                src_hbm.at[pl.ds((base + j) * _BLK, _BLK)],
                srcw.at[p], sisem[p])

        def dst_dma(j, p):
            return pltpu.make_async_copy(
                dst_hbm.at[pl.ds((base + j) * _BLK, _BLK)],
                dstw.at[p], disem[p])

        def w_dma(j, p):
            return pltpu.make_async_copy(
                w_hbm.at[pl.ds((base + j) * _BLK, _BLK)],
                ww.at[p], wsem[p])

        def gather_start(b, p):
            pltpu.async_copy(hd_hbm.at[srcw.at[p]], bufs[b], gsem[b])

        def gather_wait(b, p):
            pltpu.make_async_copy(hd_hbm.at[srcw.at[p]], bufs[b],
                                  gsem[b]).wait()

        def scatter_start(b, p):
            pltpu.async_copy(bufs[b], acc_sh.at[dstw.at[p]], ssem[b],
                             add=True)

        def scatter_wait(b, p):
            pltpu.make_async_copy(bufs[b], acc_sh.at[dstw.at[p]],
                                  ssem[b]).wait()

        @pl.loop(0, _BLK)
        def _(i):
            for q in range(128 // _LANES):
                bufs[0][i, pl.ds(q * _LANES, _LANES)] = (
                    jnp.zeros((_LANES,), jnp.float32))

        for off, nr in chunks:
            pltpu.sync_copy(bufs[0].at[pl.ds(0, nr)],
                            acc_sh.at[pl.ds(r0 + off, nr)])

        @pl.when(s == _NS - 1)
        def _():
            if ztail:
                pltpu.sync_copy(bufs[0].at[pl.ds(0, ztail)],
                                acc_sh.at[pl.ds(_NS * zc, ztail)])

        plsc.subcore_barrier()

        def scale_rows(b, wp):
            @pl.loop(0, _BLK, step=_LANES)
            def _(i0):
                wrow = ww[wp, pl.ds(i0, _LANES)]
                for u in range(_LANES):
                    wv = jnp.full((_LANES,), wrow[u])
                    for q in range(128 // _LANES):
                        sl = (i0 + u, pl.ds(q * _LANES, _LANES))
                        bufs[b][sl] = bufs[b][sl] * wv

        pltpu.sync_copy(src_hbm.at[pl.ds(base * _BLK, _BLK)], srcw.at[0])
        gather_start(0, 0)
        src_dma(1, 1).start()
        dst_dma(0, 0).start()
        w_dma(0, 0).start()

        @pl.loop(0, rpt, step=step)
        def _(j):
            for u in range(step):
                m = j + u
                b = u % nbuf
                gather_wait(b, u % 3)

                @pl.when(m + 2 < rpt)
                def _():
                    src_dma(m + 2, (u + 2) % 3).start()

                @pl.when(m + 1 < rpt)
                def _():
                    dst_dma(m + 1, (u + 1) % 4).start()
                    w_dma(m + 1, (u + 1) % 4).start()

                @pl.when(m - 2 >= 0)
                def _():
                    scatter_wait((u + 1) % 3, (u + 2) % 4)

                @pl.when(m + 1 < rpt)
                def _():
                    src_dma(m + 1, (u + 1) % 3).wait()
                    gather_start((u + 1) % 3, (u + 1) % 3)

                w_dma(m, u % 4).wait()
                scale_rows(b, u % 4)

                dst_dma(m, u % 4).wait()
                scatter_start(b, u % 4)

        scatter_wait((rpt - 2) % nbuf, (rpt - 2) % 4)
        scatter_wait((rpt - 1) % nbuf, (rpt - 1) % 4)

        plsc.subcore_barrier()
        for off, nr in chunks:
            pltpu.sync_copy(acc_sh.at[pl.ds(r0 + off, nr)],
                            bufs[0].at[pl.ds(0, nr)])
            pltpu.sync_copy(bufs[0].at[pl.ds(0, nr)],
                            out_hbm.at[c, pl.ds(r0 + off, nr)])

        @pl.when(s == _NS - 1)
        def _():
            if ztail:
                pltpu.sync_copy(acc_sh.at[pl.ds(_NS * zc, ztail)],
                                bufs[0].at[pl.ds(0, ztail)])
                pltpu.sync_copy(bufs[0].at[pl.ds(0, ztail)],
                                out_hbm.at[c, pl.ds(_NS * zc, ztail)])

    return msg_kernel(hd, src1d, dst1d, w1d)


def _tc_matmul(x, W):
    n, d = x.shape
    blk = 1000

    def body(x_ref, w_ref, o_ref):
        o_ref[...] = jnp.dot(x_ref[...], w_ref[...],
                             preferred_element_type=jnp.float32)

    return pl.pallas_call(
        body,
        grid=(n // blk,),
        in_specs=[pl.BlockSpec((blk, d), lambda i: (i, 0)),
                  pl.BlockSpec((d, d), lambda i: (0, 0))],
        out_specs=pl.BlockSpec((blk, d), lambda i: (i, 0)),
        out_shape=jax.ShapeDtypeStruct((n, d), jnp.float32),
    )(x, W)


def _tc_scale(h, degT):
    n, d = h.shape
    blk = 1000

    def body(h_ref, g_ref, o_ref):
        dg = g_ref[:, 0:1] + g_ref[:, 1:2] + 1.0
        dinv = jnp.where(dg > 0, lax.rsqrt(dg), 0.0)
        o_ref[...] = h_ref[...] * dinv

    return pl.pallas_call(
        body,
        grid=(n // blk,),
        in_specs=[pl.BlockSpec((blk, d), lambda i: (i, 0)),
                  pl.BlockSpec((blk, 2), lambda i: (i, 0))],
        out_specs=pl.BlockSpec((blk, d), lambda i: (i, 0)),
        out_shape=jax.ShapeDtypeStruct((n, d), jnp.float32),
    )(h, degT)


def _tc_combine(A2, h, degT, b):
    n, d = h.shape
    blk = 1000

    def body(a_ref, h_ref, g_ref, b_ref, o_ref):
        dg = g_ref[:, 0:1] + g_ref[:, 1:2] + 1.0
        dinv = jnp.where(dg > 0, lax.rsqrt(dg), 0.0)
        agg = (a_ref[0] + a_ref[1]) * dinv
        o_ref[...] = agg + h_ref[...] * (dinv * dinv) + b_ref[...]

    return pl.pallas_call(
        body,
        grid=(n // blk,),
        in_specs=[pl.BlockSpec((2, blk, d), lambda i: (0, i, 0)),
                  pl.BlockSpec((blk, d), lambda i: (i, 0)),
                  pl.BlockSpec((blk, 2), lambda i: (i, 0)),
                  pl.BlockSpec((1, d), lambda i: (0, 0))],
        out_specs=pl.BlockSpec((blk, d), lambda i: (i, 0)),
        out_shape=jax.ShapeDtypeStruct((n, d), jnp.float32),
    )(A2, h, degT, b)


def kernel(x, edge_index, edge_attr, W, b):
    n, d = x.shape
    e = edge_attr.shape[0]
    ei_flat = edge_index.reshape(-1)
    src = ei_flat[:e]
    dst = ei_flat[e:]
    w = edge_attr

    def pad_edges(rows):
        tgt = rows * _NW * _BLK
        padn = tgt - e
        if not padn:
            return src, dst, w
        fill = jnp.arange(padn, dtype=jnp.int32) % n
        return (jnp.concatenate([src, fill]),
                jnp.concatenate([dst, fill]),
                jnp.concatenate([w, jnp.zeros((padn,), jnp.float32)]))

    rpt0 = -(-e // (_NW * _BLK))
    _, dstd, wd = pad_edges(-(-rpt0 // 8) * 8)
    srcm, dstm, wm = pad_edges(-(-rpt0 // 12) * 12)

    deg2 = _sc_degree(dstd.reshape(-1, _BLK), wd.reshape(-1, _BLK), n)
    degT = deg2.reshape(2, n).T
    h = _tc_matmul(x, W)
    hd = _tc_scale(h, degT)
    A2 = _sc_messages(hd, srcm, dstm, wm)
    out2d = _tc_combine(A2, h, degT, b.reshape(1, d))

    seq = 8
    return jnp.transpose(out2d.reshape(n, seq, d // seq), (1, 0, 2))[None]
